# Initial kernel scaffold; baseline (speedup 1.0000x reference)
import jax
import jax.numpy as jnp
from jax.experimental import pallas as pl


def kernel(x, edge_index, W1, b1, W2, b2, W3, b3):
    raise NotImplementedError("write your pallas kernel here")


# speedup vs baseline: 1.5648x; 1.5648x over previous
import functools

import jax
import jax.numpy as jnp
from jax.experimental import pallas as pl

N = 10000
ROW_BLOCK = 400


def _mm_body(y_ref, s_ref, w_ref, b_ref, o_ref, *, leaky, out_scale):
    y = y_ref[...]
    if out_scale:
        pass
    y = y * s_ref[...]
    acc = jnp.dot(y, w_ref[...], preferred_element_type=jnp.float32)
    acc = acc + b_ref[...]
    if leaky:
        acc = jnp.where(acc > 0, acc, 0.01 * acc)
    o_ref[...] = acc


def _mm(y, s, w, b, leaky):
    K = y.shape[1]
    M = w.shape[1]
    body = functools.partial(_mm_body, leaky=leaky, out_scale=False)
    return pl.pallas_call(
        body,
        grid=(N // ROW_BLOCK,),
        in_specs=[
            pl.BlockSpec((ROW_BLOCK, K), lambda i: (i, 0)),
            pl.BlockSpec((ROW_BLOCK, 1), lambda i: (i, 0)),
            pl.BlockSpec((K, M), lambda i: (0, 0)),
            pl.BlockSpec((1, M), lambda i: (0, 0)),
        ],
        out_specs=pl.BlockSpec((ROW_BLOCK, M), lambda i: (i, 0)),
        out_shape=jax.ShapeDtypeStruct((N, M), jnp.float32),
    )(y, s, w, b)


def kernel(x, edge_index, W1, b1, W2, b2, W3, b3):
    src = edge_index[0]
    dst = edge_index[1]
    ones = jnp.ones((src.shape[0],), dtype=jnp.float32)
    deg_out = jax.ops.segment_sum(ones, src, num_segments=N)
    deg_in = jax.ops.segment_sum(ones, dst, num_segments=N)
    ns = jax.lax.rsqrt(jnp.clip(deg_out, 1.0, None))[:, None]
    nd = jax.lax.rsqrt(jnp.clip(deg_in, 1.0, None))[:, None]

    xs = x * ns
    y1 = jax.ops.segment_sum(xs[src], dst, num_segments=N)
    h1 = _mm(y1, nd, W1, b1[None, :], leaky=True)

    y2 = jax.ops.segment_sum((h1 * ns)[src], dst, num_segments=N)
    h2 = _mm(y2, nd, W2, b2[None, :], leaky=True)

    z = _mm(h2, ns, W3, jnp.zeros((1, 128), jnp.float32), leaky=False)
    y3 = jax.ops.segment_sum(z[src], dst, num_segments=N)
    return y3 * nd + b3


# speedup vs baseline: 5.1495x; 3.2908x over previous
import functools

import jax
import jax.numpy as jnp
from jax import lax
from jax.experimental import pallas as pl
from jax.experimental.pallas import tpu as pltpu
from jax.experimental.pallas import tpu_sc as plsc

N = 10000
E = 320000
NC = 2
NS = 16
NW = NC * NS
EPW = E // NW
B = 80
G = 125
HALF = 5200
ACC_H = 5376
RPS = ACC_H // NS
NPB = HALF // 400
DEG_LEN = 20480
DPS = DEG_LEN // NS
DEG_DUMMY = 20000
ROW_BLOCK = 400

_mesh = plsc.VectorSubcoreMesh(core_axis_name="c", subcore_axis_name="s",
                               num_cores=NC, num_subcores=NS)


def _deg_body(idx_hbm, ones_hbm, zer_hbm, out_hbm, idx_v, ones_v, acc_sh):
    c = lax.axis_index("c")
    s = lax.axis_index("s")
    wid = c * NS + s

    pltpu.sync_copy(ones_hbm, ones_v)
    pltpu.sync_copy(idx_hbm.at[wid], idx_v)
    pltpu.sync_copy(zer_hbm, acc_sh.at[pl.ds(s * DPS, DPS)])
    plsc.subcore_barrier()

    @pl.loop(0, 2 * G)
    def _(g):
        pltpu.sync_copy(ones_v, acc_sh.at[idx_v.at[g]], add=True)

    plsc.subcore_barrier()
    pltpu.sync_copy(acc_sh.at[pl.ds(s * DPS, DPS)],
                    out_hbm.at[c].at[pl.ds(s * DPS, DPS)])


_deg_kernel = pl.kernel(
    _deg_body,
    out_type=jax.ShapeDtypeStruct((NC, DEG_LEN), jnp.float32),
    mesh=_mesh,
    scratch_types=[
        pltpu.VMEM((2 * G, B), jnp.int32),
        pltpu.VMEM((B,), jnp.float32),
        pltpu.VMEM_SHARED((DEG_LEN,), jnp.float32),
    ],
)


def _agg_body(C, xs_hbm, sidx_hbm, didx_hbm, zeros_hbm, part_hbm,
              sidx_v, didx_v, rows_v, acc_sh, sem0, sem1):
    cid = lax.axis_index("c")
    s = lax.axis_index("s")
    wid = cid * NS + s
    sems = (sem0, sem1)

    for c in range(C):
        pltpu.sync_copy(sidx_hbm.at[c].at[wid], sidx_v)
        for p in range(2):
            pltpu.sync_copy(zeros_hbm, acc_sh.at[pl.ds(s * RPS, RPS)])
            pltpu.sync_copy(didx_hbm.at[p].at[wid], didx_v)
            plsc.subcore_barrier()

            @pl.loop(0, G - 1, step=2)
            def _(g):
                d0 = pltpu.async_copy(xs_hbm.at[sidx_v.at[g]],
                                      rows_v.at[0], sem0)
                d1 = pltpu.async_copy(xs_hbm.at[sidx_v.at[g + 1]],
                                      rows_v.at[1], sem1)
                d0.wait()
                pltpu.sync_copy(rows_v.at[0],
                                acc_sh.at[didx_v.at[g]], add=True)
                d1.wait()
                pltpu.sync_copy(rows_v.at[1],
                                acc_sh.at[didx_v.at[g + 1]], add=True)

            dt = pltpu.async_copy(xs_hbm.at[sidx_v.at[G - 1]],
                                  rows_v.at[0], sem0)
            dt.wait()
            pltpu.sync_copy(rows_v.at[0],
                            acc_sh.at[didx_v.at[G - 1]], add=True)

            plsc.subcore_barrier()
            pltpu.sync_copy(acc_sh.at[pl.ds(s * RPS, RPS)],
                            part_hbm.at[cid].at[c].at[p].at[pl.ds(s * RPS, RPS)])
            if not (c == C - 1 and p == 1):
                plsc.subcore_barrier()


def _agg(xs, sidx, didx, zeros, C):
    body = functools.partial(_agg_body, C)
    return pl.kernel(
        body,
        out_type=jax.ShapeDtypeStruct((NC, C, 2, ACC_H, 128), jnp.float32),
        mesh=_mesh,
        scratch_types=[
            pltpu.VMEM((G, B), jnp.int32),
            pltpu.VMEM((G, B), jnp.int32),
            pltpu.VMEM((2, B, 128), jnp.float32),
            pltpu.VMEM_SHARED((ACC_H, 128), jnp.float32),
            pltpu.SemaphoreType.DMA,
            pltpu.SemaphoreType.DMA,
        ],
    )(xs, sidx, didx, zeros)


def _nsd_body(deg_ref, out_ref):
    d = deg_ref[0:1, :] + deg_ref[1:2, :]
    out_ref[...] = lax.rsqrt(jnp.maximum(d, 1.0))


def _nsd(deg_part):
    return pl.pallas_call(
        _nsd_body,
        out_shape=jax.ShapeDtypeStruct((1, DEG_LEN), jnp.float32),
    )(deg_part)


def _scale_body(x_ref, s_ref, o_ref):
    o_ref[...] = x_ref[...] * s_ref[...]


def _scale(x, s):
    return pl.pallas_call(
        _scale_body,
        grid=(N // ROW_BLOCK,),
        in_specs=[
            pl.BlockSpec((ROW_BLOCK, 128), lambda i: (i, 0)),
            pl.BlockSpec((ROW_BLOCK, 1), lambda i: (i, 0)),
        ],
        out_specs=pl.BlockSpec((ROW_BLOCK, 128), lambda i: (i, 0)),
        out_shape=jax.ShapeDtypeStruct((N, 128), jnp.float32),
    )(x, s)


def _part_spec(C):
    return pl.BlockSpec((2, C, 1, ROW_BLOCK, 128),
                        lambda i, *_: (0, 0, i // NPB, i % NPB, 0))


def _mm1_body(p_ref, nd_ref, ns_ref, w_ref, b_ref, o_ref):
    y = (p_ref[0, 0, 0] + p_ref[1, 0, 0]) * nd_ref[...]
    h = jnp.dot(y, w_ref[...], preferred_element_type=jnp.float32)
    h = h + b_ref[...]
    h = jnp.where(h > 0, h, 0.01 * h)
    o_ref[0] = h * ns_ref[...]


def _mm1(part, nd, ns, W1, b1):
    return pl.pallas_call(
        _mm1_body,
        grid=(N // ROW_BLOCK, 4),
        in_specs=[
            _part_spec(1),
            pl.BlockSpec((ROW_BLOCK, 1), lambda i, c: (i, 0)),
            pl.BlockSpec((ROW_BLOCK, 1), lambda i, c: (i, 0)),
            pl.BlockSpec((128, 128), lambda i, c: (0, c)),
            pl.BlockSpec((1, 128), lambda i, c: (0, c)),
        ],
        out_specs=pl.BlockSpec((1, ROW_BLOCK, 128), lambda i, c: (c, i, 0)),
        out_shape=jax.ShapeDtypeStruct((4, N, 128), jnp.float32),
    )(part, nd, ns, W1, b1)


def _mm2_body(p_ref, nd_ref, ns_ref, w2_ref, b2_ref, w3_ref, o_ref):
    p = p_ref[...]
    y = jnp.concatenate([p[0, c, 0] + p[1, c, 0] for c in range(4)], axis=1)
    y = y * nd_ref[...]
    h = jnp.dot(y, w2_ref[...], preferred_element_type=jnp.float32)
    h = h + b2_ref[...]
    h = jnp.where(h > 0, h, 0.01 * h)
    h = h * ns_ref[...]
    o_ref[...] = jnp.dot(h, w3_ref[...], preferred_element_type=jnp.float32)


def _mm2(part, nd, ns, W2, b2, W3):
    return pl.pallas_call(
        _mm2_body,
        grid=(N // ROW_BLOCK,),
        in_specs=[
            _part_spec(4),
            pl.BlockSpec((ROW_BLOCK, 1), lambda i: (i, 0)),
            pl.BlockSpec((ROW_BLOCK, 1), lambda i: (i, 0)),
            pl.BlockSpec((512, 1024), lambda i: (0, 0)),
            pl.BlockSpec((1, 1024), lambda i: (0, 0)),
            pl.BlockSpec((1024, 128), lambda i: (0, 0)),
        ],
        out_specs=pl.BlockSpec((ROW_BLOCK, 128), lambda i: (i, 0)),
        out_shape=jax.ShapeDtypeStruct((N, 128), jnp.float32),
    )(part, nd, ns, W2, b2, W3)


def _final_body(p_ref, nd_ref, b_ref, o_ref):
    o_ref[...] = (p_ref[0, 0, 0] + p_ref[1, 0, 0]) * nd_ref[...] + b_ref[...]


def _final(part, nd, b3):
    return pl.pallas_call(
        _final_body,
        grid=(N // ROW_BLOCK,),
        in_specs=[
            _part_spec(1),
            pl.BlockSpec((ROW_BLOCK, 1), lambda i: (i, 0)),
            pl.BlockSpec((1, 128), lambda i: (0, 0)),
        ],
        out_specs=pl.BlockSpec((ROW_BLOCK, 128), lambda i: (i, 0)),
        out_shape=jax.ShapeDtypeStruct((N, 128), jnp.float32),
    )(part, nd, b3)


def kernel(x, edge_index, W1, b1, W2, b2, W3, b3):
    src = edge_index[0].reshape(NW, G, B)
    dst = edge_index[1].reshape(NW, G, B)

    deg_idx = jnp.concatenate([src, dst + N], axis=1)

    spread = jnp.bitwise_and(dst, 127)
    dlo = jnp.where(dst < HALF, dst, HALF + spread)
    dhi = jnp.where(dst >= HALF, dst - HALF, (N - HALF) + spread)
    didx = jnp.stack([dlo, dhi])
    sidx1 = src[None]
    sidx4 = src[None] + (jnp.arange(4, dtype=jnp.int32) * N)[:, None, None, None]

    deg_part = _deg_kernel(deg_idx, jnp.ones((B,), jnp.float32),
                           jnp.zeros((DPS,), jnp.float32))
    nsd = _nsd(deg_part)
    ns = nsd[0, :N].reshape(N, 1)
    nd = nsd[0, N:2 * N].reshape(N, 1)

    zeros = jnp.zeros((RPS, 128), jnp.float32)
    xs1 = _scale(x, ns)
    part1 = _agg(xs1, sidx1, didx, zeros, 1)
    xs2 = _mm1(part1, nd, ns, W1, b1.reshape(1, 512))

    part2 = _agg(xs2.reshape(4 * N, 128), sidx4, didx, zeros, 4)
    z = _mm2(part2, nd, ns, W2, b2.reshape(1, 1024), W3)

    part3 = _agg(z, sidx1, didx, zeros, 1)
    return _final(part3, nd, b3.reshape(1, 128))


# speedup vs baseline: 5.6360x; 1.0945x over previous
import functools

import jax
import jax.numpy as jnp
from jax import lax
from jax.experimental import pallas as pl
from jax.experimental.pallas import tpu as pltpu
from jax.experimental.pallas import tpu_sc as plsc

N = 10000
E = 320000
NC = 2
NS = 16
NW = NC * NS
EPW = E // NW
B = 80
G = 125
BA = 128
GA = 79
EPAD = GA * BA - EPW
ACC1 = 10112
RPS = ACC1 // NS
DEG_LEN = 20480
DPS = DEG_LEN // NS
DEG_DUMMY = 20000
ROW_BLOCK = 400

_mesh = plsc.VectorSubcoreMesh(core_axis_name="c", subcore_axis_name="s",
                               num_cores=NC, num_subcores=NS)


def _deg_body(idx_hbm, ones_hbm, zer_hbm, out_hbm, idx_v, ones_v, acc_sh):
    c = lax.axis_index("c")
    s = lax.axis_index("s")
    wid = c * NS + s

    pltpu.sync_copy(ones_hbm, ones_v)
    pltpu.sync_copy(idx_hbm.at[wid], idx_v)
    pltpu.sync_copy(zer_hbm, acc_sh.at[pl.ds(s * DPS, DPS)])
    plsc.subcore_barrier()

    @pl.loop(0, 2 * G)
    def _(g):
        pltpu.sync_copy(ones_v, acc_sh.at[idx_v.at[g]], add=True)

    plsc.subcore_barrier()
    pltpu.sync_copy(acc_sh.at[pl.ds(s * DPS, DPS)],
                    out_hbm.at[c].at[pl.ds(s * DPS, DPS)])


_deg_kernel = pl.kernel(
    _deg_body,
    out_type=jax.ShapeDtypeStruct((NC, DEG_LEN), jnp.float32),
    mesh=_mesh,
    scratch_types=[
        pltpu.VMEM((2 * G, B), jnp.int32),
        pltpu.VMEM((B,), jnp.float32),
        pltpu.VMEM_SHARED((DEG_LEN,), jnp.float32),
    ],
)


def _agg_body(C, xs_hbm, sidx_hbm, didx_hbm, zeros_hbm, part_hbm,
              sidx_v, didx_v, rows_v, acc_sh, sem0, sem1, sem2, sem3):
    cid = lax.axis_index("c")
    s = lax.axis_index("s")
    wid = cid * NS + s

    for c in range(C):
        pltpu.sync_copy(sidx_hbm.at[c].at[wid], sidx_v)
        pltpu.sync_copy(zeros_hbm, acc_sh.at[pl.ds(s * RPS, RPS)])
        plsc.subcore_barrier()

        @pl.loop(0, GA - 1, step=2)
        def _(g):
            d0 = pltpu.async_copy(xs_hbm.at[sidx_v.at[g]], rows_v.at[0], sem0)
            e0 = pltpu.async_copy(didx_hbm.at[wid].at[g],
                                  didx_v.at[pl.ds(0, 1)], sem2)
            d1 = pltpu.async_copy(xs_hbm.at[sidx_v.at[g + 1]],
                                  rows_v.at[1], sem1)
            e1 = pltpu.async_copy(didx_hbm.at[wid].at[g + 1],
                                  didx_v.at[pl.ds(1, 1)], sem3)
            d0.wait()
            e0.wait()
            pltpu.sync_copy(rows_v.at[0], acc_sh.at[didx_v.at[0]], add=True)
            d1.wait()
            e1.wait()
            pltpu.sync_copy(rows_v.at[1], acc_sh.at[didx_v.at[1]], add=True)

        dt = pltpu.async_copy(xs_hbm.at[sidx_v.at[GA - 1]], rows_v.at[0], sem0)
        et = pltpu.async_copy(didx_hbm.at[wid].at[GA - 1],
                              didx_v.at[pl.ds(0, 1)], sem2)
        dt.wait()
        et.wait()
        pltpu.sync_copy(rows_v.at[0], acc_sh.at[didx_v.at[0]], add=True)

        plsc.subcore_barrier()
        pltpu.sync_copy(acc_sh.at[pl.ds(s * RPS, RPS)],
                        part_hbm.at[cid].at[c].at[pl.ds(s * RPS, RPS)])
        if c + 1 < C:
            plsc.subcore_barrier()


def _agg(xs, sidx, didx, zeros, C):
    body = functools.partial(_agg_body, C)
    return pl.kernel(
        body,
        out_type=jax.ShapeDtypeStruct((NC, C, ACC1, 128), jnp.float32),
        mesh=_mesh,
        scratch_types=[
            pltpu.VMEM((GA, BA), jnp.int32),
            pltpu.VMEM((2, BA), jnp.int32),
            pltpu.VMEM((2, BA, 128), jnp.float32),
            pltpu.VMEM_SHARED((ACC1, 128), jnp.float32),
            pltpu.SemaphoreType.DMA,
            pltpu.SemaphoreType.DMA,
            pltpu.SemaphoreType.DMA,
            pltpu.SemaphoreType.DMA,
        ],
    )(xs, sidx, didx, zeros)


def _nsd_body(deg_ref, out_ref):
    d = deg_ref[0:1, :] + deg_ref[1:2, :]
    out_ref[...] = lax.rsqrt(jnp.maximum(d, 1.0))


def _nsd(deg_part):
    return pl.pallas_call(
        _nsd_body,
        out_shape=jax.ShapeDtypeStruct((1, DEG_LEN), jnp.float32),
    )(deg_part)


def _scale_body(x_ref, s_ref, o_ref):
    o_ref[...] = (x_ref[...] * s_ref[...]).astype(o_ref.dtype)


def _scale(x, s):
    return pl.pallas_call(
        _scale_body,
        grid=(N // ROW_BLOCK,),
        in_specs=[
            pl.BlockSpec((ROW_BLOCK, 128), lambda i: (i, 0)),
            pl.BlockSpec((ROW_BLOCK, 1), lambda i: (i, 0)),
        ],
        out_specs=pl.BlockSpec((ROW_BLOCK, 128), lambda i: (i, 0)),
        out_shape=jax.ShapeDtypeStruct((N, 128), jnp.float32),
    )(x, s)


def _part_spec(C):
    return pl.BlockSpec((2, C, ROW_BLOCK, 128),
                        lambda i, *_: (0, 0, i, 0))


def _mm1_body(p_ref, nd_ref, ns_ref, w_ref, b_ref, o_ref):
    y = (p_ref[0, 0] + p_ref[1, 0]) * nd_ref[...]
    h = jnp.dot(y, w_ref[...], preferred_element_type=jnp.float32)
    h = h + b_ref[...]
    h = jnp.where(h > 0, h, 0.01 * h)
    o_ref[0] = (h * ns_ref[...]).astype(o_ref.dtype)


def _mm1(part, nd, ns, W1, b1):
    return pl.pallas_call(
        _mm1_body,
        grid=(N // ROW_BLOCK, 4),
        in_specs=[
            _part_spec(1),
            pl.BlockSpec((ROW_BLOCK, 1), lambda i, c: (i, 0)),
            pl.BlockSpec((ROW_BLOCK, 1), lambda i, c: (i, 0)),
            pl.BlockSpec((128, 128), lambda i, c: (0, c)),
            pl.BlockSpec((1, 128), lambda i, c: (0, c)),
        ],
        out_specs=pl.BlockSpec((1, ROW_BLOCK, 128), lambda i, c: (c, i, 0)),
        out_shape=jax.ShapeDtypeStruct((4, N, 128), jnp.float32),
    )(part, nd, ns, W1, b1)


def _mm2_body(p_ref, nd_ref, ns_ref, w2_ref, b2_ref, w3_ref, o_ref):
    p = p_ref[...]
    y = jnp.concatenate([p[0, c] + p[1, c] for c in range(4)], axis=1)
    y = y * nd_ref[...]
    h = jnp.dot(y, w2_ref[...], preferred_element_type=jnp.float32)
    h = h + b2_ref[...]
    h = jnp.where(h > 0, h, 0.01 * h)
    h = h * ns_ref[...]
    o_ref[...] = jnp.dot(h, w3_ref[...],
                         preferred_element_type=jnp.float32).astype(o_ref.dtype)


def _mm2(part, nd, ns, W2, b2, W3):
    return pl.pallas_call(
        _mm2_body,
        grid=(N // ROW_BLOCK,),
        in_specs=[
            _part_spec(4),
            pl.BlockSpec((ROW_BLOCK, 1), lambda i: (i, 0)),
            pl.BlockSpec((ROW_BLOCK, 1), lambda i: (i, 0)),
            pl.BlockSpec((512, 1024), lambda i: (0, 0)),
            pl.BlockSpec((1, 1024), lambda i: (0, 0)),
            pl.BlockSpec((1024, 128), lambda i: (0, 0)),
        ],
        out_specs=pl.BlockSpec((ROW_BLOCK, 128), lambda i: (i, 0)),
        out_shape=jax.ShapeDtypeStruct((N, 128), jnp.float32),
    )(part, nd, ns, W2, b2, W3)


def _final_body(p_ref, nd_ref, b_ref, o_ref):
    o_ref[...] = (p_ref[0, 0] + p_ref[1, 0]) * nd_ref[...] + b_ref[...]


def _final(part, nd, b3):
    return pl.pallas_call(
        _final_body,
        grid=(N // ROW_BLOCK,),
        in_specs=[
            _part_spec(1),
            pl.BlockSpec((ROW_BLOCK, 1), lambda i: (i, 0)),
            pl.BlockSpec((1, 128), lambda i: (0, 0)),
        ],
        out_specs=pl.BlockSpec((ROW_BLOCK, 128), lambda i: (i, 0)),
        out_shape=jax.ShapeDtypeStruct((N, 128), jnp.float32),
    )(part, nd, b3)


def kernel(x, edge_index, W1, b1, W2, b2, W3, b3):
    src2 = edge_index[0].reshape(NW, EPW)
    dst2 = edge_index[1].reshape(NW, EPW)

    deg_idx = jnp.concatenate([src2.reshape(NW, G, B),
                               dst2.reshape(NW, G, B) + N], axis=1)

    srcp = jnp.pad(src2, ((0, 0), (0, EPAD))).reshape(NW, GA, BA)
    didx = jnp.pad(dst2, ((0, 0), (0, EPAD)),
                   constant_values=N).reshape(NW, GA, 1, BA)
    sidx1 = srcp[None]
    sidx4 = srcp[None] + (jnp.arange(4, dtype=jnp.int32) * N)[:, None, None, None]

    deg_part = _deg_kernel(deg_idx, jnp.ones((B,), jnp.float32),
                           jnp.zeros((DPS,), jnp.float32))
    nsd = _nsd(deg_part)
    ns = nsd[0, :N].reshape(N, 1)
    nd = nsd[0, N:2 * N].reshape(N, 1)

    zeros = jnp.zeros((RPS, 128), jnp.float32)
    xs1 = _scale(x, ns)
    part1 = _agg(xs1, sidx1, didx, zeros, 1)
    xs2 = _mm1(part1, nd, ns, W1, b1.reshape(1, 512))

    part2 = _agg(xs2.reshape(4 * N, 128), sidx4, didx, zeros, 4)
    z = _mm2(part2, nd, ns, W2, b2.reshape(1, 1024), W3)

    part3 = _agg(z, sidx1, didx, zeros, 1)
    return _final(part3, nd, b3.reshape(1, 128))


# speedup vs baseline: 5.7392x; 1.0183x over previous
import functools

import jax
import jax.numpy as jnp
from jax import lax
from jax.experimental import pallas as pl
from jax.experimental.pallas import tpu as pltpu
from jax.experimental.pallas import tpu_sc as plsc

N = 10000
E = 320000
NC = 2
NS = 16
NW = NC * NS
EPW = E // NW
B = 80
G = 125
BA = 128
GA = 79
EPAD = GA * BA - EPW
ACC1 = 10112
RPS = ACC1 // NS
DEG_LEN = 20480
DPS = DEG_LEN // NS
DEG_DUMMY = 20000
ROW_BLOCK = 400

_mesh = plsc.VectorSubcoreMesh(core_axis_name="c", subcore_axis_name="s",
                               num_cores=NC, num_subcores=NS)


def _deg_body(idx_hbm, ones_hbm, zer_hbm, out_hbm, idx_v, ones_v, acc_sh):
    c = lax.axis_index("c")
    s = lax.axis_index("s")
    wid = c * NS + s

    pltpu.sync_copy(ones_hbm, ones_v)
    pltpu.sync_copy(idx_hbm.at[wid], idx_v)
    pltpu.sync_copy(zer_hbm, acc_sh.at[pl.ds(s * DPS, DPS)])
    plsc.subcore_barrier()

    @pl.loop(0, 2 * G)
    def _(g):
        pltpu.sync_copy(ones_v, acc_sh.at[idx_v.at[g]], add=True)

    plsc.subcore_barrier()
    pltpu.sync_copy(acc_sh.at[pl.ds(s * DPS, DPS)],
                    out_hbm.at[c].at[pl.ds(s * DPS, DPS)])


_deg_kernel = pl.kernel(
    _deg_body,
    out_type=jax.ShapeDtypeStruct((NC, DEG_LEN), jnp.float32),
    mesh=_mesh,
    scratch_types=[
        pltpu.VMEM((2 * G, B), jnp.int32),
        pltpu.VMEM((B,), jnp.float32),
        pltpu.VMEM_SHARED((DEG_LEN,), jnp.float32),
    ],
)


def _agg_body(C, xs_hbm, sidx_hbm, didx_hbm, zeros_hbm, part_hbm,
              sidx_v, didx_v, rows_v, acc_sh,
              g0, g1, e0, e1, s0, s1):
    cid = lax.axis_index("c")
    s = lax.axis_index("s")
    wid = cid * NS + s
    gsem = (g0, g1)
    esem = (e0, e1)
    ssem = (s0, s1)

    def fetch(g, b):
        pltpu.async_copy(xs_hbm.at[sidx_v.at[g]], rows_v.at[b], gsem[b])
        pltpu.async_copy(didx_hbm.at[wid].at[g], didx_v.at[pl.ds(b, 1)],
                         esem[b])

    def wait_fetch(b):
        pltpu.make_async_copy(xs_hbm.at[sidx_v.at[0]], rows_v.at[b],
                              gsem[b]).wait()
        pltpu.make_async_copy(didx_hbm.at[wid].at[0], didx_v.at[pl.ds(b, 1)],
                              esem[b]).wait()

    def scatter(b):
        return pltpu.async_copy(rows_v.at[b], acc_sh.at[didx_v.at[b]],
                                ssem[b], add=True)

    for c in range(C):
        pltpu.sync_copy(sidx_hbm.at[c].at[wid], sidx_v)
        pltpu.sync_copy(zeros_hbm, acc_sh.at[pl.ds(s * RPS, RPS)])
        plsc.subcore_barrier()

        fetch(0, 0)
        fetch(1, 1)

        @pl.loop(0, GA - 1, step=2)
        def _(g):
            wait_fetch(0)
            sa0 = scatter(0)
            wait_fetch(1)
            sa1 = scatter(1)
            sa0.wait()

            @pl.when(g + 2 < GA)
            def _():
                fetch(g + 2, 0)

            sa1.wait()

            @pl.when(g + 3 < GA)
            def _():
                fetch(g + 3, 1)

        wait_fetch(0)
        scatter(0).wait()

        plsc.subcore_barrier()
        pltpu.sync_copy(acc_sh.at[pl.ds(s * RPS, RPS)],
                        part_hbm.at[cid].at[c].at[pl.ds(s * RPS, RPS)])
        if c + 1 < C:
            plsc.subcore_barrier()


def _agg(xs, sidx, didx, zeros, C):
    body = functools.partial(_agg_body, C)
    return pl.kernel(
        body,
        out_type=jax.ShapeDtypeStruct((NC, C, ACC1, 128), jnp.float32),
        mesh=_mesh,
        scratch_types=[
            pltpu.VMEM((GA, BA), jnp.int32),
            pltpu.VMEM((2, BA), jnp.int32),
            pltpu.VMEM((2, BA, 128), jnp.float32),
            pltpu.VMEM_SHARED((ACC1, 128), jnp.float32),
            pltpu.SemaphoreType.DMA,
            pltpu.SemaphoreType.DMA,
            pltpu.SemaphoreType.DMA,
            pltpu.SemaphoreType.DMA,
            pltpu.SemaphoreType.DMA,
            pltpu.SemaphoreType.DMA,
        ],
    )(xs, sidx, didx, zeros)


def _nsd_body(deg_ref, out_ref):
    d = deg_ref[0:1, :] + deg_ref[1:2, :]
    out_ref[...] = lax.rsqrt(jnp.maximum(d, 1.0))


def _nsd(deg_part):
    return pl.pallas_call(
        _nsd_body,
        out_shape=jax.ShapeDtypeStruct((1, DEG_LEN), jnp.float32),
    )(deg_part)


def _scale_body(x_ref, s_ref, o_ref):
    o_ref[...] = (x_ref[...] * s_ref[...]).astype(o_ref.dtype)


def _scale(x, s):
    return pl.pallas_call(
        _scale_body,
        grid=(N // ROW_BLOCK,),
        in_specs=[
            pl.BlockSpec((ROW_BLOCK, 128), lambda i: (i, 0)),
            pl.BlockSpec((ROW_BLOCK, 1), lambda i: (i, 0)),
        ],
        out_specs=pl.BlockSpec((ROW_BLOCK, 128), lambda i: (i, 0)),
        out_shape=jax.ShapeDtypeStruct((N, 128), jnp.float32),
    )(x, s)


def _part_spec(C):
    return pl.BlockSpec((2, C, ROW_BLOCK, 128),
                        lambda i, *_: (0, 0, i, 0))


def _mm1_body(p_ref, nd_ref, ns_ref, w_ref, b_ref, o_ref):
    y = (p_ref[0, 0] + p_ref[1, 0]) * nd_ref[...]
    h = jnp.dot(y, w_ref[...], preferred_element_type=jnp.float32)
    h = h + b_ref[...]
    h = jnp.where(h > 0, h, 0.01 * h)
    o_ref[0] = (h * ns_ref[...]).astype(o_ref.dtype)


def _mm1(part, nd, ns, W1, b1):
    return pl.pallas_call(
        _mm1_body,
        grid=(N // ROW_BLOCK, 4),
        in_specs=[
            _part_spec(1),
            pl.BlockSpec((ROW_BLOCK, 1), lambda i, c: (i, 0)),
            pl.BlockSpec((ROW_BLOCK, 1), lambda i, c: (i, 0)),
            pl.BlockSpec((128, 128), lambda i, c: (0, c)),
            pl.BlockSpec((1, 128), lambda i, c: (0, c)),
        ],
        out_specs=pl.BlockSpec((1, ROW_BLOCK, 128), lambda i, c: (c, i, 0)),
        out_shape=jax.ShapeDtypeStruct((4, N, 128), jnp.float32),
    )(part, nd, ns, W1, b1)


def _mm2_body(p_ref, nd_ref, ns_ref, w2_ref, b2_ref, w3_ref, o_ref):
    p = p_ref[...]
    y = jnp.concatenate([p[0, c] + p[1, c] for c in range(4)], axis=1)
    y = y * nd_ref[...]
    h = jnp.dot(y, w2_ref[...], preferred_element_type=jnp.float32)
    h = h + b2_ref[...]
    h = jnp.where(h > 0, h, 0.01 * h)
    h = h * ns_ref[...]
    o_ref[...] = jnp.dot(h, w3_ref[...],
                         preferred_element_type=jnp.float32).astype(o_ref.dtype)


def _mm2(part, nd, ns, W2, b2, W3):
    return pl.pallas_call(
        _mm2_body,
        grid=(N // ROW_BLOCK,),
        in_specs=[
            _part_spec(4),
            pl.BlockSpec((ROW_BLOCK, 1), lambda i: (i, 0)),
            pl.BlockSpec((ROW_BLOCK, 1), lambda i: (i, 0)),
            pl.BlockSpec((512, 1024), lambda i: (0, 0)),
            pl.BlockSpec((1, 1024), lambda i: (0, 0)),
            pl.BlockSpec((1024, 128), lambda i: (0, 0)),
        ],
        out_specs=pl.BlockSpec((ROW_BLOCK, 128), lambda i: (i, 0)),
        out_shape=jax.ShapeDtypeStruct((N, 128), jnp.float32),
    )(part, nd, ns, W2, b2, W3)


def _final_body(p_ref, nd_ref, b_ref, o_ref):
    o_ref[...] = (p_ref[0, 0] + p_ref[1, 0]) * nd_ref[...] + b_ref[...]


def _final(part, nd, b3):
    return pl.pallas_call(
        _final_body,
        grid=(N // ROW_BLOCK,),
        in_specs=[
            _part_spec(1),
            pl.BlockSpec((ROW_BLOCK, 1), lambda i: (i, 0)),
            pl.BlockSpec((1, 128), lambda i: (0, 0)),
        ],
        out_specs=pl.BlockSpec((ROW_BLOCK, 128), lambda i: (i, 0)),
        out_shape=jax.ShapeDtypeStruct((N, 128), jnp.float32),
    )(part, nd, b3)


def kernel(x, edge_index, W1, b1, W2, b2, W3, b3):
    src2 = edge_index[0].reshape(NW, EPW)
    dst2 = edge_index[1].reshape(NW, EPW)

    deg_idx = jnp.concatenate([src2.reshape(NW, G, B),
                               dst2.reshape(NW, G, B) + N], axis=1)

    srcp = jnp.pad(src2, ((0, 0), (0, EPAD))).reshape(NW, GA, BA)
    didx = jnp.pad(dst2, ((0, 0), (0, EPAD)),
                   constant_values=N).reshape(NW, GA, 1, BA)
    sidx1 = srcp[None]
    sidx4 = srcp[None] + (jnp.arange(4, dtype=jnp.int32) * N)[:, None, None, None]

    deg_part = _deg_kernel(deg_idx, jnp.ones((B,), jnp.float32),
                           jnp.zeros((DPS,), jnp.float32))
    nsd = _nsd(deg_part)
    ns = nsd[0, :N].reshape(N, 1)
    nd = nsd[0, N:2 * N].reshape(N, 1)

    zeros = jnp.zeros((RPS, 128), jnp.float32)
    xs1 = _scale(x, ns)
    part1 = _agg(xs1, sidx1, didx, zeros, 1)
    xs2 = _mm1(part1, nd, ns, W1, b1.reshape(1, 512))

    part2 = _agg(xs2.reshape(4 * N, 128), sidx4, didx, zeros, 4)
    z = _mm2(part2, nd, ns, W2, b2.reshape(1, 1024), W3)

    part3 = _agg(z, sidx1, didx, zeros, 1)
    return _final(part3, nd, b3.reshape(1, 128))
